# speedup vs baseline: 1.2754x; 1.2754x over previous
import functools

import jax
import jax.numpy as jnp
from jax import lax
from jax.experimental import pallas as pl
from jax.experimental.pallas import tpu as pltpu
from jax.experimental.pallas import tpu_sc as plsc

D = 768
NW = 32
CHUNK = 112


@functools.lru_cache(maxsize=None)
def _build(n_total: int):
    b_per_w = n_total // NW
    n_chunks = b_per_w // CHUNK
    assert b_per_w % CHUNK == 0 and n_total % NW == 0

    mesh = plsc.VectorSubcoreMesh(core_axis_name="c", subcore_axis_name="s")

    @functools.partial(
        pl.kernel,
        mesh=mesh,
        out_type=jax.ShapeDtypeStruct((n_total, D), jnp.float32),
        scratch_types=[
            pltpu.VMEM((b_per_w,), jnp.int32),
            pltpu.VMEM((CHUNK, D), jnp.float32),
            pltpu.SemaphoreType.DMA,
        ],
    )
    def gather_kernel(ids_hbm, table_hbm, out_hbm, idx_v, rows_v, sem):
        wid = lax.axis_index("s") * 2 + lax.axis_index("c")
        base = wid * b_per_w
        pltpu.sync_copy(ids_hbm.at[pl.ds(base, b_per_w)], idx_v)

        @pl.loop(0, n_chunks)
        def _(c):
            off = c * CHUNK
            pltpu.async_copy(
                table_hbm.at[idx_v.at[pl.ds(off, CHUNK)]], rows_v, sem
            ).wait()
            pltpu.sync_copy(rows_v, out_hbm.at[pl.ds(base + off, CHUNK)])

    return gather_kernel


def kernel(input_ids, token_embedding):
    b, s = input_ids.shape
    flat_ids = input_ids.reshape(-1).astype(jnp.int32)
    out = _build(flat_ids.shape[0])(flat_ids, token_embedding)
    return out.reshape(b, s, D)


# speedup vs baseline: 1.2968x; 1.0168x over previous
import functools

import jax
import jax.numpy as jnp
from jax import lax
from jax.experimental import pallas as pl
from jax.experimental.pallas import tpu as pltpu
from jax.experimental.pallas import tpu_sc as plsc

D = 768
NW = 32
CHUNK = 56


@functools.lru_cache(maxsize=None)
def _build(n_total: int):
    b_per_w = n_total // NW
    n_chunks = b_per_w // CHUNK
    assert b_per_w % CHUNK == 0 and n_total % NW == 0 and n_chunks % 2 == 0

    mesh = plsc.VectorSubcoreMesh(core_axis_name="c", subcore_axis_name="s")

    @functools.partial(
        pl.kernel,
        mesh=mesh,
        out_type=jax.ShapeDtypeStruct((n_total, D), jnp.float32),
        scratch_types=[
            pltpu.VMEM((b_per_w,), jnp.int32),
            pltpu.VMEM((CHUNK, D), jnp.float32),
            pltpu.VMEM((CHUNK, D), jnp.float32),
            pltpu.SemaphoreType.DMA,
            pltpu.SemaphoreType.DMA,
        ],
    )
    def gather_kernel(ids_hbm, table_hbm, out_hbm, idx_v, buf0, buf1, sem0, sem1):
        wid = lax.axis_index("s") * 2 + lax.axis_index("c")
        base = wid * b_per_w
        pltpu.sync_copy(ids_hbm.at[pl.ds(base, b_per_w)], idx_v)

        bufs = (buf0, buf1)
        sems = (sem0, sem1)

        pltpu.async_copy(table_hbm.at[idx_v.at[pl.ds(0, CHUNK)]], buf0, sem0)
        pltpu.async_copy(table_hbm.at[idx_v.at[pl.ds(CHUNK, CHUNK)]], buf1, sem1)

        @pl.loop(0, n_chunks, step=2)
        def _(c):
            for b in range(2):
                cg = c + b
                off = cg * CHUNK
                pltpu.make_async_copy(
                    table_hbm.at[idx_v.at[pl.ds(0, CHUNK)]], bufs[b], sems[b]
                ).wait()
                pltpu.sync_copy(bufs[b], out_hbm.at[pl.ds(base + off, CHUNK)])

                @pl.when(cg + 2 < n_chunks)
                def _():
                    pltpu.async_copy(
                        table_hbm.at[idx_v.at[pl.ds(off + 2 * CHUNK, CHUNK)]],
                        bufs[b],
                        sems[b],
                    )

    return gather_kernel


def kernel(input_ids, token_embedding):
    b, s = input_ids.shape
    flat_ids = input_ids.reshape(-1).astype(jnp.int32)
    out = _build(flat_ids.shape[0])(flat_ids, token_embedding)
    return out.reshape(b, s, D)


# speedup vs baseline: 1.8798x; 1.4496x over previous
import functools

import jax
import jax.numpy as jnp
from jax import lax
from jax.experimental import pallas as pl
from jax.experimental.pallas import tpu as pltpu
from jax.experimental.pallas import tpu_sc as plsc

D = 768
NW = 32


@functools.lru_cache(maxsize=None)
def _build(batch: int, seqpad: int):
    seq_per_w = batch // NW
    assert batch % NW == 0 and seq_per_w % 2 == 0 and seqpad % 8 == 0

    mesh = plsc.VectorSubcoreMesh(core_axis_name="c", subcore_axis_name="s")

    @functools.partial(
        pl.kernel,
        mesh=mesh,
        out_type=jax.ShapeDtypeStruct((batch, seqpad, D), jnp.float32),
        scratch_types=[
            pltpu.VMEM((seq_per_w * seqpad,), jnp.int32),
            pltpu.VMEM((seqpad, D), jnp.float32),
            pltpu.VMEM((seqpad, D), jnp.float32),
            pltpu.SemaphoreType.DMA,
            pltpu.SemaphoreType.DMA,
        ],
    )
    def gather_kernel(ids_hbm, table_hbm, out_hbm, idx_v, buf0, buf1, sem0, sem1):
        wid = lax.axis_index("s") * 2 + lax.axis_index("c")
        seq0 = wid * seq_per_w
        pltpu.sync_copy(
            ids_hbm.at[pl.ds(seq0 * seqpad, seq_per_w * seqpad)], idx_v
        )

        bufs = (buf0, buf1)
        sems = (sem0, sem1)

        pltpu.async_copy(table_hbm.at[idx_v.at[pl.ds(0, seqpad)]], buf0, sem0)
        pltpu.async_copy(table_hbm.at[idx_v.at[pl.ds(seqpad, seqpad)]], buf1, sem1)

        @pl.loop(0, seq_per_w, step=2)
        def _(s):
            for b in range(2):
                sg = s + b
                pltpu.make_async_copy(
                    table_hbm.at[idx_v.at[pl.ds(0, seqpad)]], bufs[b], sems[b]
                ).wait()
                pltpu.sync_copy(bufs[b], out_hbm.at[seq0 + sg])

                @pl.when(sg + 2 < seq_per_w)
                def _():
                    pltpu.async_copy(
                        table_hbm.at[idx_v.at[pl.ds((sg + 2) * seqpad, seqpad)]],
                        bufs[b],
                        sems[b],
                    )

    return gather_kernel


def kernel(input_ids, token_embedding):
    b, s = input_ids.shape
    spad = (s + 7) // 8 * 8
    ids = input_ids.astype(jnp.int32)
    ids = jnp.concatenate([ids, ids[:, : spad - s]], axis=1).reshape(-1)
    out = _build(b, spad)(ids, token_embedding)
    return out[:, :s, :]


# speedup vs baseline: 2.0122x; 1.0704x over previous
import functools

import jax
import jax.numpy as jnp
from jax import lax
from jax.experimental import pallas as pl
from jax.experimental.pallas import tpu as pltpu
from jax.experimental.pallas import tpu_sc as plsc

D = 768
NW = 32


@functools.lru_cache(maxsize=None)
def _build(batch: int, seqlen: int, seqpad: int):
    seq_per_w = batch // NW
    assert batch % NW == 0 and seq_per_w % 2 == 0 and seqpad % 8 == 0

    mesh = plsc.VectorSubcoreMesh(core_axis_name="c", subcore_axis_name="s")

    @functools.partial(
        pl.kernel,
        mesh=mesh,
        out_type=jax.ShapeDtypeStruct((batch, seqlen, D), jnp.float32),
        compiler_params=pltpu.CompilerParams(use_tc_tiling_on_sc=True),
        scratch_types=[
            pltpu.VMEM((seq_per_w * seqpad,), jnp.int32),
            pltpu.VMEM((seqlen, D), jnp.float32),
            pltpu.VMEM((seqlen, D), jnp.float32),
            pltpu.SemaphoreType.DMA,
            pltpu.SemaphoreType.DMA,
        ],
    )
    def gather_kernel(ids_hbm, table_hbm, out_hbm, idx_v, buf0, buf1, sem0, sem1):
        wid = lax.axis_index("s") * 2 + lax.axis_index("c")
        seq0 = wid * seq_per_w
        pltpu.sync_copy(
            ids_hbm.at[pl.ds(seq0 * seqpad, seq_per_w * seqpad)], idx_v
        )

        bufs = (buf0, buf1)
        sems = (sem0, sem1)

        pltpu.async_copy(table_hbm.at[idx_v.at[pl.ds(0, seqlen)]], buf0, sem0)
        pltpu.async_copy(table_hbm.at[idx_v.at[pl.ds(seqpad, seqlen)]], buf1, sem1)

        @pl.loop(0, seq_per_w, step=2)
        def _(s):
            for b in range(2):
                sg = s + b
                pltpu.make_async_copy(
                    table_hbm.at[idx_v.at[pl.ds(0, seqlen)]], bufs[b], sems[b]
                ).wait()
                pltpu.sync_copy(bufs[b], out_hbm.at[seq0 + sg])

                @pl.when(sg + 2 < seq_per_w)
                def _():
                    pltpu.async_copy(
                        table_hbm.at[idx_v.at[pl.ds((sg + 2) * seqpad, seqlen)]],
                        bufs[b],
                        sems[b],
                    )

    return gather_kernel


def kernel(input_ids, token_embedding):
    b, s = input_ids.shape
    spad = (s + 7) // 8 * 8
    ids = input_ids.astype(jnp.int32)
    ids = jnp.concatenate([ids, ids[:, : spad - s]], axis=1).reshape(-1)
    return _build(b, s, spad)(ids, token_embedding)


# speedup vs baseline: 3.7759x; 1.8765x over previous
import functools

import jax
import jax.numpy as jnp
from jax import lax
from jax.experimental import pallas as pl
from jax.experimental.pallas import tpu as pltpu
from jax.experimental.pallas import tpu_sc as plsc

D = 768
NW = 32


@functools.lru_cache(maxsize=None)
def _build(batch: int, seqlen: int):
    b_per_w = batch // NW
    assert batch % NW == 0 and b_per_w % 8 == 0

    mesh = plsc.VectorSubcoreMesh(core_axis_name="c", subcore_axis_name="s")

    n_main = seqlen - (seqlen % 2)

    @functools.partial(
        pl.kernel,
        mesh=mesh,
        out_type=jax.ShapeDtypeStruct((seqlen, batch, D), jnp.float32),
        compiler_params=pltpu.CompilerParams(use_tc_tiling_on_sc=True),
        scratch_types=[
            pltpu.VMEM((seqlen * b_per_w,), jnp.int32),
            pltpu.VMEM((b_per_w, D), jnp.float32),
            pltpu.VMEM((b_per_w, D), jnp.float32),
            pltpu.SemaphoreType.DMA,
            pltpu.SemaphoreType.DMA,
        ],
    )
    def gather_kernel(ids_hbm, table_hbm, out_hbm, idx_v, buf0, buf1, sem0, sem1):
        wid = lax.axis_index("s") * 2 + lax.axis_index("c")
        pltpu.sync_copy(
            ids_hbm.at[pl.ds(wid * seqlen * b_per_w, seqlen * b_per_w)], idx_v
        )

        bufs = (buf0, buf1)
        sems = (sem0, sem1)

        def gather(s, b):
            pltpu.async_copy(
                table_hbm.at[idx_v.at[pl.ds(s * b_per_w, b_per_w)]],
                bufs[b],
                sems[b],
            )

        def wait(b):
            pltpu.make_async_copy(
                table_hbm.at[idx_v.at[pl.ds(0, b_per_w)]], bufs[b], sems[b]
            ).wait()

        def write(s, b):
            pltpu.sync_copy(bufs[b], out_hbm.at[s, pl.ds(wid * b_per_w, b_per_w)])

        gather(0, 0)
        gather(1, 1)

        @pl.loop(0, n_main, step=2)
        def _(s):
            for b in range(2):
                sg = s + b
                wait(b)
                write(sg, b)

                @pl.when(sg + 2 < seqlen)
                def _():
                    gather(sg + 2, b)

        if n_main < seqlen:
            wait(0)
            write(seqlen - 1, 0)

    return gather_kernel


def kernel(input_ids, token_embedding):
    b, s = input_ids.shape
    ids = (
        input_ids.astype(jnp.int32)
        .reshape(NW, b // NW, s)
        .transpose(0, 2, 1)
        .reshape(-1)
    )
    out = _build(b, s)(ids, token_embedding)
    return out.transpose(1, 0, 2)
